# Initial kernel scaffold; baseline (speedup 1.0000x reference)
import jax
import jax.numpy as jnp
from jax.experimental import pallas as pl


def kernel(input, code, codebook):
    raise NotImplementedError("write your pallas kernel here")


# speedup vs baseline: 25.3086x; 25.3086x over previous
import functools

import jax
import jax.numpy as jnp
from jax import lax
from jax.experimental import pallas as pl
from jax.experimental.pallas import tpu as pltpu
from jax.experimental.pallas import tpu_sc as plsc


def _table_body(code_ref, cb_ref, out_ref, *, num_codebook, num_codeword):
    x = code_ref[...]
    cb = cb_ref[...]
    probs = []
    for b in range(num_codebook):
        g = x[:, b * num_codeword:(b + 1) * num_codeword]
        m = jnp.max(g, axis=1, keepdims=True)
        e = jnp.exp(g - m)
        probs.append(e / jnp.sum(e, axis=1, keepdims=True))
    p = jnp.concatenate(probs, axis=1)
    out_ref[...] = jnp.dot(p, cb, preferred_element_type=jnp.float32)


def _build_table(code2d, cb2d, num_codebook, num_codeword, block_rows):
    v, f = code2d.shape
    d = cb2d.shape[1]
    assert v % block_rows == 0
    grid = (v // block_rows,)
    return pl.pallas_call(
        functools.partial(_table_body, num_codebook=num_codebook,
                          num_codeword=num_codeword),
        grid=grid,
        in_specs=[
            pl.BlockSpec((block_rows, f), lambda i: (i, 0)),
            pl.BlockSpec((f, d), lambda i: (0, 0)),
        ],
        out_specs=pl.BlockSpec((block_rows, d), lambda i: (i, 0)),
        out_shape=jax.ShapeDtypeStruct((v, d), jnp.float32),
    )(code2d, cb2d)


def _make_sc_gather(v, d, b):
    info = plsc.get_sparse_core_info()
    nc, ns = info.num_cores, info.num_subcores
    nw = nc * ns
    assert b % (8 * nw) == 0
    b_per_w = b // nw
    mesh = plsc.VectorSubcoreMesh(core_axis_name="c", subcore_axis_name="s")

    @functools.partial(
        pl.kernel,
        mesh=mesh,
        out_type=jax.ShapeDtypeStruct((b, d), jnp.float32),
        scratch_types=[
            pltpu.VMEM((b_per_w,), jnp.int32),
            pltpu.VMEM((b_per_w, d), jnp.float32),
            pltpu.SemaphoreType.DMA,
        ],
        compiler_params=pltpu.CompilerParams(use_tc_tiling_on_sc=False),
    )
    def gather(table_hbm, idx_hbm, out_hbm, idx_v, rows_v, sem):
        wid = lax.axis_index("s") * nc + lax.axis_index("c")
        base = wid * b_per_w
        pltpu.sync_copy(idx_hbm.at[pl.ds(base, b_per_w)], idx_v)
        pltpu.async_copy(table_hbm.at[idx_v], rows_v, sem).wait()
        pltpu.sync_copy(rows_v, out_hbm.at[pl.ds(base, b_per_w)])

    return gather


def kernel(input, code, codebook):
    batch, w = input.shape
    v, num_codebook, num_codeword = code.shape
    d = codebook.shape[-1]
    f = num_codebook * num_codeword

    code2d = code.reshape(v, f)
    cb2d = codebook.reshape(f, d)
    table = _build_table(code2d, cb2d, num_codebook, num_codeword,
                         block_rows=2000)

    idx = input.reshape(-1).astype(jnp.int32)
    out = _make_sc_gather(v, d, idx.shape[0])(table, idx)
    return out.reshape(batch, w, d)
